# baseline (speedup 1.0000x reference)
import functools

import jax
import jax.numpy as jnp
from jax import lax
from jax.experimental import pallas as pl
from jax.experimental.pallas import tpu as pltpu
from jax.experimental.pallas import tpu_sc as plsc

_NC = 2
_NS = 16
_NW = _NC * _NS
_L = 16

_B = 16384
_T = 1000
_B_PER_W = _B // _NW


def _body(ts_hbm, alphas_hbm, out_hbm, vals_v, idx_v, out_v, sem_t, sem_i):
    wid = lax.axis_index("s") * _NC + lax.axis_index("c")
    base = wid * _B_PER_W
    pltpu.sync_copy(ts_hbm.at[pl.ds(base, _B_PER_W)], idx_v)
    pltpu.async_copy(alphas_hbm.at[idx_v], vals_v, sem_t).wait()
    for i in range(_B_PER_W // _L):
        a = vals_v[pl.ds(i * _L, _L)]
        out_v[pl.ds(i * _L, _L)] = a / (1.0 - a)
    pltpu.sync_copy(out_v, out_hbm.at[pl.ds(base, _B_PER_W)])


@jax.jit
def kernel(timesteps, alphas_cumprod):
    run = pl.kernel(
        _body,
        out_type=jax.ShapeDtypeStruct((_B,), jnp.float32),
        mesh=plsc.VectorSubcoreMesh(core_axis_name="c", subcore_axis_name="s"),
        scratch_types=[
            pltpu.VMEM((_B_PER_W,), jnp.float32),
            pltpu.VMEM((_B_PER_W,), jnp.int32),
            pltpu.VMEM((_B_PER_W,), jnp.float32),
            pltpu.SemaphoreType.DMA,
            pltpu.SemaphoreType.DMA,
        ],
        compiler_params=pltpu.CompilerParams(
            needs_layout_passes=False, skip_device_barrier=True
        ),
    )
    return run(timesteps.astype(jnp.int32), alphas_cumprod)


# speedup vs baseline: 1.4036x; 1.4036x over previous
import functools

import jax
import jax.numpy as jnp
from jax import lax
from jax.experimental import pallas as pl
from jax.experimental.pallas import tpu as pltpu
from jax.experimental.pallas import tpu_sc as plsc

_NC = 2
_NS = 16
_NW = _NC * _NS
_L = 16

_B = 16384
_T = 1000
_B_PER_W = _B // _NW


def _body(ts_hbm, alphas_hbm, out_hbm, table_v, idx_v, out_v, sem_t, sem_i):
    wid = lax.axis_index("s") * _NC + lax.axis_index("c")
    base = wid * _B_PER_W
    cp_t = pltpu.async_copy(alphas_hbm, table_v, sem_t)
    cp_i = pltpu.async_copy(ts_hbm.at[pl.ds(base, _B_PER_W)], idx_v, sem_i)
    cp_t.wait()
    cp_i.wait()
    half = _B_PER_W // 2
    for i in range(half // _L):
        idx = idx_v[pl.ds(i * _L, _L)]
        a = plsc.load_gather(table_v, [idx])
        out_v[pl.ds(i * _L, _L)] = a / (1.0 - a)
    cp_o = pltpu.async_copy(
        out_v.at[pl.ds(0, half)], out_hbm.at[pl.ds(base, half)], sem_i
    )
    for i in range(half // _L, _B_PER_W // _L):
        idx = idx_v[pl.ds(i * _L, _L)]
        a = plsc.load_gather(table_v, [idx])
        out_v[pl.ds(i * _L, _L)] = a / (1.0 - a)
    cp_o.wait()
    pltpu.sync_copy(
        out_v.at[pl.ds(half, half)], out_hbm.at[pl.ds(base + half, half)]
    )


@jax.jit
def kernel(timesteps, alphas_cumprod):
    run = pl.kernel(
        _body,
        out_type=jax.ShapeDtypeStruct((_B,), jnp.float32),
        mesh=plsc.VectorSubcoreMesh(core_axis_name="c", subcore_axis_name="s"),
        scratch_types=[
            pltpu.VMEM((_T,), jnp.float32),
            pltpu.VMEM((_B_PER_W,), jnp.int32),
            pltpu.VMEM((_B_PER_W,), jnp.float32),
            pltpu.SemaphoreType.DMA,
            pltpu.SemaphoreType.DMA,
        ],
        compiler_params=pltpu.CompilerParams(
            needs_layout_passes=False, skip_device_barrier=True
        ),
    )
    return run(timesteps.astype(jnp.int32), alphas_cumprod)


# speedup vs baseline: 1.4251x; 1.0153x over previous
import functools

import jax
import jax.numpy as jnp
from jax import lax
from jax.experimental import pallas as pl
from jax.experimental.pallas import tpu as pltpu
from jax.experimental.pallas import tpu_sc as plsc

_NC = 2
_NS = 16
_NW = _NC * _NS
_L = 16

_B = 16384
_T = 1000
_B_PER_W = _B // _NW


def _body(ts_hbm, alphas_hbm, out_hbm, table_v, idx_v, out_v, sem_t, sem_i):
    wid = lax.axis_index("s") * _NC + lax.axis_index("c")
    base = wid * _B_PER_W
    cp_t = pltpu.async_copy(alphas_hbm, table_v, sem_t)
    cp_i = pltpu.async_copy(ts_hbm.at[pl.ds(base, _B_PER_W)], idx_v, sem_i)
    cp_t.wait()
    cp_i.wait()
    def step(i, carry):
        off = i * _L
        idx = idx_v[pl.ds(off, _L)]
        a = plsc.load_gather(table_v, [idx])
        out_v[pl.ds(off, _L)] = a / (1.0 - a)
        return carry

    lax.fori_loop(0, _B_PER_W // _L, step, 0)
    pltpu.sync_copy(out_v, out_hbm.at[pl.ds(base, _B_PER_W)])


@jax.jit
def kernel(timesteps, alphas_cumprod):
    run = pl.kernel(
        _body,
        out_type=jax.ShapeDtypeStruct((_B,), jnp.float32),
        mesh=plsc.VectorSubcoreMesh(core_axis_name="c", subcore_axis_name="s"),
        scratch_types=[
            pltpu.VMEM((_T,), jnp.float32),
            pltpu.VMEM((_B_PER_W,), jnp.int32),
            pltpu.VMEM((_B_PER_W,), jnp.float32),
            pltpu.SemaphoreType.DMA,
            pltpu.SemaphoreType.DMA,
        ],
        compiler_params=pltpu.CompilerParams(
            needs_layout_passes=False, skip_device_barrier=True
        ),
    )
    return run(timesteps.astype(jnp.int32), alphas_cumprod)
